# baseline (speedup 1.0000x reference)
import dataclasses
import functools

import jax
import jax.numpy as jnp
from jax import lax
from jax.experimental import pallas as pl
from jax.experimental.pallas import tpu as pltpu
from jax.experimental.pallas import tpu_sc as plsc

N = 10000
E = 320000
D = 128

NC = 2
NS = 16
NW = NC * NS
EP = 327680
SBE = 320
SBS = 8
SLEN = 40
NSB = EP // (NW * SBE)
EPW = EP // NW
IROWS = EP // SLEN
NPAD = 10240
RPT = NPAD // NS
ZROWS = 128

def _sc_compiler_params():
    cp = pltpu.CompilerParams()
    if "needs_layout_passes" in pltpu.CompilerParams.__dataclass_fields__:
        cp = dataclasses.replace(cp, needs_layout_passes=False)
    return cp


def _sc_gather_body(src2_hbm, src1_hbm, w_hbm, cj_hbm, g1_hbm, g2_hbm,
                    idx2_v, idx1a_v, idx1b_v, rows_v, g2ba_v, g2bb_v, cj_v,
                    si, s1, sg, sw, sw2):
    wid = lax.axis_index("s") * NC + lax.axis_index("c")
    edge_base = wid * EPW
    pltpu.sync_copy(cj_hbm, cj_v)

    @pl.loop(0, EPW // 80)
    def _(b):
        off = edge_base + b * 80
        pltpu.sync_copy(src1_hbm.at[pl.ds(off, 80)], idx1a_v)
        c1 = pltpu.async_copy(w_hbm.at[idx1a_v], rows_v, sg.at[0])
        @pl.loop(0, 5)
        def _(j):
            iv = idx1a_v[pl.ds(j * 16, 16)]
            g2ba_v[pl.ds(j * 16, 16)] = plsc.load_gather(cj_v, [iv])
        c1.wait()
        pltpu.sync_copy(rows_v, g1_hbm.at[pl.ds(off, 80)])
        pltpu.sync_copy(g2ba_v, g2_hbm.at[pl.ds(off, 80)])


@functools.cache
def _build_sc_gather():
    mesh = plsc.VectorSubcoreMesh(
        core_axis_name="c", subcore_axis_name="s",
        num_cores=NC, num_subcores=NS)
    return pl.kernel(
        _sc_gather_body,
        out_type=[jax.ShapeDtypeStruct((EP, D), jnp.float32),
                  jax.ShapeDtypeStruct((EP,), jnp.float32)],
        mesh=mesh,
        scratch_types=[pltpu.VMEM((2, SBS, SLEN), jnp.int32),
                       pltpu.VMEM((80,), jnp.int32),
                       pltpu.VMEM((80,), jnp.int32),
                       pltpu.VMEM((80, D), jnp.float32),
                       pltpu.VMEM((80,), jnp.float32),
                       pltpu.VMEM((80,), jnp.float32),
                       pltpu.VMEM((N,), jnp.float32),
                       pltpu.SemaphoreType.DMA((2,)),
                       pltpu.SemaphoreType.DMA((2,)),
                       pltpu.SemaphoreType.DMA((2,)),
                       pltpu.SemaphoreType.DMA((2,)),
                       pltpu.SemaphoreType.DMA((2,))],
        compiler_params=_sc_compiler_params(),
    )


BE = 512
NBE = E // BE


def _tc_main_body(x_ref, attn_ref, g1_ref, g2_ref, wT_ref, pw_ref, rw_ref, m_ref):
    x = x_ref[...]
    rf = jnp.dot(x, wT_ref[...], preferred_element_type=jnp.float32)
    pa_lin = jnp.sum(x * pw_ref[...], axis=1, keepdims=True)
    ra_lin = jnp.sum(x * rw_ref[...], axis=1, keepdims=True)
    pa = 1.0 / (1.0 + jnp.exp(-pa_lin))
    ra = 1.0 / (1.0 + jnp.exp(-ra_lin))
    cj_src = g2_ref[...]
    m_ref[...] = (g1_ref[...] * pa + rf * (ra * attn_ref[...])) * cj_src


_tc_main = pl.pallas_call(
    _tc_main_body,
    grid=(NBE,),
    in_specs=[
        pl.BlockSpec((BE, D), lambda i: (i, 0)),
        pl.BlockSpec((BE, 1), lambda i: (i, 0)),
        pl.BlockSpec((BE, D), lambda i: (i, 0)),
        pl.BlockSpec((BE, 1), lambda i: (i, 0)),
        pl.BlockSpec((D, D), lambda i: (0, 0)),
        pl.BlockSpec((1, D), lambda i: (0, 0)),
        pl.BlockSpec((1, D), lambda i: (0, 0)),
    ],
    out_specs=pl.BlockSpec((BE, D), lambda i: (i, 0)),
    out_shape=jax.ShapeDtypeStruct((EP, D), jnp.float32),
)


def _sc_scatter_body(dst2_hbm, m_hbm, z_hbm, out_hbm,
                     idxa_v, idxb_v, rowsa_v, rowsb_v, acc_sh,
                     sia, sib, sra, srb, ssa, ssb):
    cid = lax.axis_index("c")
    sid = lax.axis_index("s")
    wid = sid * NC + cid
    row_base = wid * (EPW // SLEN)
    edge_base = wid * EPW

    @pl.loop(0, RPT // ZROWS)
    def _(k):
        pltpu.sync_copy(z_hbm, acc_sh.at[pl.ds(sid * RPT + k * ZROWS, ZROWS)])
    plsc.subcore_barrier()

    @pl.loop(0, NSB)
    def _(sb):
        r = row_base + sb * SBS
        e = edge_base + sb * SBE
        ia = pltpu.async_copy(dst2_hbm.at[pl.ds(r, SBS)], idxa_v, sia)
        ca = pltpu.async_copy(m_hbm.at[pl.ds(e, SBE)], rowsa_v, sra)
        ia.wait()
        ca.wait()
        sca = [pltpu.async_copy(rowsa_v.at[pl.ds(k * SLEN, SLEN)],
                                acc_sh.at[idxa_v.at[k]], ssa, add=True)
               for k in range(SBS)]
        for c in sca:
            c.wait()

    plsc.subcore_barrier()

    @pl.loop(0, RPT // ZROWS)
    def _(k):
        r = sid * RPT + k * ZROWS
        pltpu.sync_copy(acc_sh.at[pl.ds(r, ZROWS)], out_hbm.at[cid].at[pl.ds(r, ZROWS)])


@functools.cache
def _build_sc_scatter():
    mesh = plsc.VectorSubcoreMesh(
        core_axis_name="c", subcore_axis_name="s",
        num_cores=NC, num_subcores=NS)
    return pl.kernel(
        _sc_scatter_body,
        out_type=jax.ShapeDtypeStruct((NC, NPAD, D), jnp.float32),
        mesh=mesh,
        scratch_types=[pltpu.VMEM((SBS, SLEN), jnp.int32),
                       pltpu.VMEM((SBS, SLEN), jnp.int32),
                       pltpu.VMEM((SBE, D), jnp.float32),
                       pltpu.VMEM((SBE, D), jnp.float32),
                       pltpu.VMEM_SHARED((NPAD, D), jnp.float32),
                       pltpu.SemaphoreType.DMA,
                       pltpu.SemaphoreType.DMA,
                       pltpu.SemaphoreType.DMA,
                       pltpu.SemaphoreType.DMA,
                       pltpu.SemaphoreType.DMA,
                       pltpu.SemaphoreType.DMA],
    )


BN = 1000
NBN = N // BN


def _tc_combine_body(p_ref, ci_ref, o_ref):
    o_ref[...] = (p_ref[0] + p_ref[1]) * ci_ref[...]


_tc_combine = pl.pallas_call(
    _tc_combine_body,
    grid=(NBN,),
    in_specs=[
        pl.BlockSpec((NC, BN, D), lambda i: (0, i, 0)),
        pl.BlockSpec((BN, 1), lambda i: (i, 0)),
    ],
    out_specs=pl.BlockSpec((BN, D), lambda i: (i, 0)),
    out_shape=jax.ShapeDtypeStruct((N, D), jnp.float32),
)


def kernel(edge_index, attn, review_feat, cj, ci, weight, prob_score_w,
           review_score_w, review_w):
    src = jnp.concatenate([edge_index[0], jnp.zeros((EP - E,), jnp.int32)])
    dst = jnp.concatenate(
        [edge_index[1], N + (jnp.arange(EP - E, dtype=jnp.int32) % (NPAD - N))])
    attn2 = attn.reshape(E, 1)
    zeros = jnp.zeros((ZROWS, D), jnp.float32)

    g1, g2 = _build_sc_gather()(src.reshape(IROWS, SLEN), src, weight,
                                cj.reshape(N))
    m = _tc_main(review_feat, attn2, g1, g2[:E].reshape(E, 1), review_w.T,
                 prob_score_w, review_score_w)
    parts = _build_sc_scatter()(dst.reshape(IROWS, SLEN), m, zeros)
    return _tc_combine(parts, ci)


# speedup vs baseline: 1.3166x; 1.3166x over previous
import dataclasses
import functools

import jax
import jax.numpy as jnp
from jax import lax
from jax.experimental import pallas as pl
from jax.experimental.pallas import tpu as pltpu
from jax.experimental.pallas import tpu_sc as plsc

N = 10000
E = 320000
D = 128

NC = 2
NS = 16
NW = NC * NS
EP = 327680
SBE = 320
SBS = 8
SLEN = 40
NSB = EP // (NW * SBE)
EPW = EP // NW
IROWS = EP // SLEN
NPAD = 10240
RPT = NPAD // NS
ZROWS = 128

def _sc_compiler_params():
    cp = pltpu.CompilerParams()
    if "needs_layout_passes" in pltpu.CompilerParams.__dataclass_fields__:
        cp = dataclasses.replace(cp, needs_layout_passes=False)
    return cp


def _sc_gather_body(src1_hbm, w_hbm, cj_hbm, g1_hbm, g2_hbm,
                    idx1a_v, rows_v, g2ba_v, cj_v, sg):
    wid = lax.axis_index("s") * NC + lax.axis_index("c")
    edge_base = wid * (E // NW)
    pltpu.sync_copy(cj_hbm, cj_v)

    @pl.loop(0, E // NW // 80)
    def _(b):
        off = edge_base + b * 80
        pltpu.sync_copy(src1_hbm.at[pl.ds(off, 80)], idx1a_v)
        c1 = pltpu.async_copy(w_hbm.at[idx1a_v], rows_v, sg.at[0])
        @pl.loop(0, 5)
        def _(j):
            iv = idx1a_v[pl.ds(j * 16, 16)]
            g2ba_v[pl.ds(j * 16, 16)] = plsc.load_gather(cj_v, [iv])
        c1.wait()
        pltpu.sync_copy(rows_v, g1_hbm.at[pl.ds(off, 80)])
        pltpu.sync_copy(g2ba_v, g2_hbm.at[pl.ds(off, 80)])


@functools.cache
def _build_sc_gather():
    mesh = plsc.VectorSubcoreMesh(
        core_axis_name="c", subcore_axis_name="s",
        num_cores=NC, num_subcores=NS)
    return pl.kernel(
        _sc_gather_body,
        out_type=[jax.ShapeDtypeStruct((E, D), jnp.float32),
                  jax.ShapeDtypeStruct((E,), jnp.float32)],
        mesh=mesh,
        scratch_types=[pltpu.VMEM((80,), jnp.int32),
                       pltpu.VMEM((80, D), jnp.float32),
                       pltpu.VMEM((80,), jnp.float32),
                       pltpu.VMEM((N,), jnp.float32),
                       pltpu.SemaphoreType.DMA((2,))],
        compiler_params=_sc_compiler_params(),
    )


BE = 512
NBE = E // BE


def _tc_main_body(x_ref, attn_ref, g1_ref, g2_ref, wT_ref, pw_ref, rw_ref, m_ref):
    x = x_ref[...]
    rf = jnp.dot(x, wT_ref[...], preferred_element_type=jnp.float32)
    pa_lin = jnp.sum(x * pw_ref[...], axis=1, keepdims=True)
    ra_lin = jnp.sum(x * rw_ref[...], axis=1, keepdims=True)
    pa = 1.0 / (1.0 + jnp.exp(-pa_lin))
    ra = 1.0 / (1.0 + jnp.exp(-ra_lin))
    cj_src = g2_ref[...]
    m_ref[...] = (g1_ref[...] * pa + rf * (ra * attn_ref[...])) * cj_src


_tc_main = pl.pallas_call(
    _tc_main_body,
    grid=(NBE,),
    in_specs=[
        pl.BlockSpec((BE, D), lambda i: (i, 0)),
        pl.BlockSpec((BE, 1), lambda i: (i, 0)),
        pl.BlockSpec((BE, D), lambda i: (i, 0)),
        pl.BlockSpec((BE, 1), lambda i: (i, 0)),
        pl.BlockSpec((D, D), lambda i: (0, 0)),
        pl.BlockSpec((1, D), lambda i: (0, 0)),
        pl.BlockSpec((1, D), lambda i: (0, 0)),
    ],
    out_specs=pl.BlockSpec((BE, D), lambda i: (i, 0)),
    out_shape=jax.ShapeDtypeStruct((EP, D), jnp.float32),
)


def _sc_scatter_body(dst2_hbm, m_hbm, z_hbm, out_hbm,
                     idxa_v, idxb_v, rowsa_v, rowsb_v, acc_sh,
                     sia, sib, sra, srb, ssa, ssb):
    cid = lax.axis_index("c")
    sid = lax.axis_index("s")
    wid = sid * NC + cid
    row_base = wid * (EPW // SLEN)
    edge_base = wid * EPW

    @pl.loop(0, RPT // ZROWS)
    def _(k):
        pltpu.sync_copy(z_hbm, acc_sh.at[pl.ds(sid * RPT + k * ZROWS, ZROWS)])
    plsc.subcore_barrier()

    @pl.loop(0, NSB)
    def _(sb):
        r = row_base + sb * SBS
        e = edge_base + sb * SBE
        ia = pltpu.async_copy(dst2_hbm.at[pl.ds(r, SBS)], idxa_v, sia)
        ca = pltpu.async_copy(m_hbm.at[pl.ds(e, SBE)], rowsa_v, sra)
        ia.wait()
        ca.wait()
        sca = [pltpu.async_copy(rowsa_v.at[pl.ds(k * SLEN, SLEN)],
                                acc_sh.at[idxa_v.at[k]], ssa, add=True)
               for k in range(SBS)]
        for c in sca:
            c.wait()

    plsc.subcore_barrier()

    @pl.loop(0, RPT // ZROWS)
    def _(k):
        r = sid * RPT + k * ZROWS
        pltpu.sync_copy(acc_sh.at[pl.ds(r, ZROWS)], out_hbm.at[cid].at[pl.ds(r, ZROWS)])


@functools.cache
def _build_sc_scatter():
    mesh = plsc.VectorSubcoreMesh(
        core_axis_name="c", subcore_axis_name="s",
        num_cores=NC, num_subcores=NS)
    return pl.kernel(
        _sc_scatter_body,
        out_type=jax.ShapeDtypeStruct((NC, NPAD, D), jnp.float32),
        mesh=mesh,
        scratch_types=[pltpu.VMEM((SBS, SLEN), jnp.int32),
                       pltpu.VMEM((SBS, SLEN), jnp.int32),
                       pltpu.VMEM((SBE, D), jnp.float32),
                       pltpu.VMEM((SBE, D), jnp.float32),
                       pltpu.VMEM_SHARED((NPAD, D), jnp.float32),
                       pltpu.SemaphoreType.DMA,
                       pltpu.SemaphoreType.DMA,
                       pltpu.SemaphoreType.DMA,
                       pltpu.SemaphoreType.DMA,
                       pltpu.SemaphoreType.DMA,
                       pltpu.SemaphoreType.DMA],
    )


BN = 1000
NBN = N // BN


def _tc_combine_body(p_ref, ci_ref, o_ref):
    o_ref[...] = (p_ref[0] + p_ref[1]) * ci_ref[...]


_tc_combine = pl.pallas_call(
    _tc_combine_body,
    grid=(NBN,),
    in_specs=[
        pl.BlockSpec((NC, BN, D), lambda i: (0, i, 0)),
        pl.BlockSpec((BN, 1), lambda i: (i, 0)),
    ],
    out_specs=pl.BlockSpec((BN, D), lambda i: (i, 0)),
    out_shape=jax.ShapeDtypeStruct((N, D), jnp.float32),
)


def kernel(edge_index, attn, review_feat, cj, ci, weight, prob_score_w,
           review_score_w, review_w):
    dst = jnp.concatenate(
        [edge_index[1], N + (jnp.arange(EP - E, dtype=jnp.int32) % (NPAD - N))])
    attn2 = attn.reshape(E, 1)
    zeros = jnp.zeros((ZROWS, D), jnp.float32)

    g1, g2 = _build_sc_gather()(edge_index[0], weight, cj.reshape(N))
    m = _tc_main(review_feat, attn2, g1, g2.reshape(E, 1), review_w.T,
                 prob_score_w, review_score_w)
    parts = _build_sc_scatter()(dst.reshape(IROWS, SLEN), m, zeros)
    return _tc_combine(parts, ci)


# speedup vs baseline: 1.4624x; 1.1107x over previous
import dataclasses
import functools

import jax
import jax.numpy as jnp
from jax import lax
from jax.experimental import pallas as pl
from jax.experimental.pallas import tpu as pltpu
from jax.experimental.pallas import tpu_sc as plsc

N = 10000
E = 320000
D = 128

NC = 2
NS = 16
NW = NC * NS
EP = 327680
SBE = 320
SBS = 8
SLEN = 40
NSB = EP // (NW * SBE)
EPW = EP // NW
IROWS = EP // SLEN
NPAD = 10240
RPT = NPAD // NS
ZROWS = 128

def _sc_compiler_params():
    cp = pltpu.CompilerParams()
    if "needs_layout_passes" in pltpu.CompilerParams.__dataclass_fields__:
        cp = dataclasses.replace(cp, needs_layout_passes=False)
    return cp


GSB = 400
GNS = E // NW // GSB


def _sc_gather_body(src1_hbm, w_hbm, cj_hbm, g1_hbm, g2_hbm,
                    idxa_v, idxb_v, rowsa_v, rowsb_v, g2ba_v, g2bb_v, cj_v,
                    si, sg, sw, sw2):
    idx_v = [idxa_v, idxb_v]
    rows_v = [rowsa_v, rowsb_v]
    g2b_v = [g2ba_v, g2bb_v]
    wid = lax.axis_index("s") * NC + lax.axis_index("c")
    edge_base = wid * (E // NW)
    pltpu.sync_copy(cj_hbm, cj_v)

    ic = [None, None]
    wb1 = [None, None]
    wb2 = [None, None]
    ic[0] = pltpu.async_copy(src1_hbm.at[pl.ds(edge_base, GSB)], idx_v[0],
                             si.at[0])
    for sb in range(GNS):
        s = sb % 2
        n = (sb + 1) % 2
        if sb + 1 < GNS:
            e = edge_base + (sb + 1) * GSB
            ic[n] = pltpu.async_copy(src1_hbm.at[pl.ds(e, GSB)], idx_v[n],
                                     si.at[n])
        ic[s].wait()
        if wb1[s] is not None:
            wb1[s].wait()
            wb2[s].wait()
        gathers = [
            pltpu.async_copy(w_hbm.at[idx_v[s].at[pl.ds(k * 80, 80)]],
                             rows_v[s].at[pl.ds(k * 80, 80)], sg.at[s])
            for k in range(GSB // 80)
        ]
        i1 = idx_v[s]
        gb = g2b_v[s]

        @pl.loop(0, GSB // 16)
        def _(j):
            iv = i1[pl.ds(j * 16, 16)]
            gb[pl.ds(j * 16, 16)] = plsc.load_gather(cj_v, [iv])
        for g in gathers:
            g.wait()
        e = edge_base + sb * GSB
        wb1[s] = pltpu.async_copy(rows_v[s], g1_hbm.at[pl.ds(e, GSB)], sw.at[s])
        wb2[s] = pltpu.async_copy(g2b_v[s], g2_hbm.at[pl.ds(e, GSB)], sw2.at[s])
    for s in range(2):
        if wb1[s] is not None:
            wb1[s].wait()
            wb2[s].wait()


@functools.cache
def _build_sc_gather():
    mesh = plsc.VectorSubcoreMesh(
        core_axis_name="c", subcore_axis_name="s",
        num_cores=NC, num_subcores=NS)
    return pl.kernel(
        _sc_gather_body,
        out_type=[jax.ShapeDtypeStruct((E, D), jnp.float32),
                  jax.ShapeDtypeStruct((E,), jnp.float32)],
        mesh=mesh,
        scratch_types=[pltpu.VMEM((GSB,), jnp.int32),
                       pltpu.VMEM((GSB,), jnp.int32),
                       pltpu.VMEM((GSB, D), jnp.float32),
                       pltpu.VMEM((GSB, D), jnp.float32),
                       pltpu.VMEM((GSB,), jnp.float32),
                       pltpu.VMEM((GSB,), jnp.float32),
                       pltpu.VMEM((N,), jnp.float32),
                       pltpu.SemaphoreType.DMA((2,)),
                       pltpu.SemaphoreType.DMA((2,)),
                       pltpu.SemaphoreType.DMA((2,)),
                       pltpu.SemaphoreType.DMA((2,))],
        compiler_params=_sc_compiler_params(),
    )


BE = 512
NBE = E // BE


def _tc_main_body(x_ref, attn_ref, g1_ref, g2_ref, wT_ref, pw_ref, rw_ref, m_ref):
    x = x_ref[...]
    rf = jnp.dot(x, wT_ref[...], preferred_element_type=jnp.float32)
    pa_lin = jnp.sum(x * pw_ref[...], axis=1, keepdims=True)
    ra_lin = jnp.sum(x * rw_ref[...], axis=1, keepdims=True)
    pa = 1.0 / (1.0 + jnp.exp(-pa_lin))
    ra = 1.0 / (1.0 + jnp.exp(-ra_lin))
    cj_src = g2_ref[...]
    m_ref[...] = (g1_ref[...] * pa + rf * (ra * attn_ref[...])) * cj_src


_tc_main = pl.pallas_call(
    _tc_main_body,
    grid=(NBE,),
    in_specs=[
        pl.BlockSpec((BE, D), lambda i: (i, 0)),
        pl.BlockSpec((BE, 1), lambda i: (i, 0)),
        pl.BlockSpec((BE, D), lambda i: (i, 0)),
        pl.BlockSpec((BE, 1), lambda i: (i, 0)),
        pl.BlockSpec((D, D), lambda i: (0, 0)),
        pl.BlockSpec((1, D), lambda i: (0, 0)),
        pl.BlockSpec((1, D), lambda i: (0, 0)),
    ],
    out_specs=pl.BlockSpec((BE, D), lambda i: (i, 0)),
    out_shape=jax.ShapeDtypeStruct((EP, D), jnp.float32),
)


def _sc_scatter_body(dst2_hbm, m_hbm, z_hbm, out_hbm,
                     idxa_v, idxb_v, rowsa_v, rowsb_v, acc_sh,
                     sia, sib, sra, srb, ssa, ssb):
    cid = lax.axis_index("c")
    sid = lax.axis_index("s")
    wid = sid * NC + cid
    row_base = wid * (EPW // SLEN)
    edge_base = wid * EPW

    @pl.loop(0, RPT // ZROWS)
    def _(k):
        pltpu.sync_copy(z_hbm, acc_sh.at[pl.ds(sid * RPT + k * ZROWS, ZROWS)])
    plsc.subcore_barrier()

    @pl.loop(0, NSB)
    def _(sb):
        r = row_base + sb * SBS
        e = edge_base + sb * SBE
        ia = pltpu.async_copy(dst2_hbm.at[pl.ds(r, SBS)], idxa_v, sia)
        ca = pltpu.async_copy(m_hbm.at[pl.ds(e, SBE)], rowsa_v, sra)
        ia.wait()
        ca.wait()
        sca = [pltpu.async_copy(rowsa_v.at[pl.ds(k * SLEN, SLEN)],
                                acc_sh.at[idxa_v.at[k]], ssa, add=True)
               for k in range(SBS)]
        for c in sca:
            c.wait()

    plsc.subcore_barrier()

    @pl.loop(0, RPT // ZROWS)
    def _(k):
        r = sid * RPT + k * ZROWS
        pltpu.sync_copy(acc_sh.at[pl.ds(r, ZROWS)], out_hbm.at[cid].at[pl.ds(r, ZROWS)])


@functools.cache
def _build_sc_scatter():
    mesh = plsc.VectorSubcoreMesh(
        core_axis_name="c", subcore_axis_name="s",
        num_cores=NC, num_subcores=NS)
    return pl.kernel(
        _sc_scatter_body,
        out_type=jax.ShapeDtypeStruct((NC, NPAD, D), jnp.float32),
        mesh=mesh,
        scratch_types=[pltpu.VMEM((SBS, SLEN), jnp.int32),
                       pltpu.VMEM((SBS, SLEN), jnp.int32),
                       pltpu.VMEM((SBE, D), jnp.float32),
                       pltpu.VMEM((SBE, D), jnp.float32),
                       pltpu.VMEM_SHARED((NPAD, D), jnp.float32),
                       pltpu.SemaphoreType.DMA,
                       pltpu.SemaphoreType.DMA,
                       pltpu.SemaphoreType.DMA,
                       pltpu.SemaphoreType.DMA,
                       pltpu.SemaphoreType.DMA,
                       pltpu.SemaphoreType.DMA],
    )


BN = 1000
NBN = N // BN


def _tc_combine_body(p_ref, ci_ref, o_ref):
    o_ref[...] = (p_ref[0] + p_ref[1]) * ci_ref[...]


_tc_combine = pl.pallas_call(
    _tc_combine_body,
    grid=(NBN,),
    in_specs=[
        pl.BlockSpec((NC, BN, D), lambda i: (0, i, 0)),
        pl.BlockSpec((BN, 1), lambda i: (i, 0)),
    ],
    out_specs=pl.BlockSpec((BN, D), lambda i: (i, 0)),
    out_shape=jax.ShapeDtypeStruct((N, D), jnp.float32),
)


def kernel(edge_index, attn, review_feat, cj, ci, weight, prob_score_w,
           review_score_w, review_w):
    dst = jnp.concatenate(
        [edge_index[1], N + (jnp.arange(EP - E, dtype=jnp.int32) % (NPAD - N))])
    attn2 = attn.reshape(E, 1)
    zeros = jnp.zeros((ZROWS, D), jnp.float32)

    g1, g2 = _build_sc_gather()(edge_index[0], weight, cj.reshape(N))
    m = _tc_main(review_feat, attn2, g1, g2.reshape(E, 1), review_w.T,
                 prob_score_w, review_score_w)
    parts = _build_sc_scatter()(dst.reshape(IROWS, SLEN), m, zeros)
    return _tc_combine(parts, ci)
